# baseline (speedup 1.0000x reference)
import functools

import jax
import jax.numpy as jnp
from jax import lax
from jax.experimental import pallas as pl
from jax.experimental.pallas import tpu as pltpu
from jax.experimental.pallas import tpu_sc as plsc

_N = 1024
_D = 64
_NPRED = 131072
_HI = lax.Precision.HIGHEST


def _gcn_body(adj_ref, x_ref, w1_ref, b1_ref, w2_ref, b2_ref,
              w3_ref, b3_ref, out_ref, a_scr):
    adj = adj_ref[...]
    a_scr[...] = ((adj + adj.T) > 0).astype(jnp.float32)
    a = a_scr[...]
    h = x_ref[...]
    for w_ref, b_ref, relu in ((w1_ref, b1_ref, True),
                               (w2_ref, b2_ref, True),
                               (w3_ref, b3_ref, False)):
        hw = jnp.dot(h, w_ref[...],
                     preferred_element_type=jnp.float32)
        o = jnp.dot(a, hw,
                    preferred_element_type=jnp.float32) + b_ref[...]
        h = jnp.maximum(o, 0.0) if relu else o
    out_ref[...] = h


def _gcn_stack(adj, x, w1, b1, w2, b2, w3p, b3p):
    return pl.pallas_call(
        _gcn_body,
        out_shape=jax.ShapeDtypeStruct((_N, 128), jnp.float32),
        scratch_shapes=[pltpu.VMEM((_N, _N), jnp.float32)],
    )(adj, x, w1, b1, w2, b2, w3p, b3p)


def _sc_gather_mul(table, idx2d):
    nrow, ch = idx2d.shape
    b = nrow * ch // 2
    dp = table.shape[1]
    info = plsc.get_sparse_core_info()
    nw = info.num_cores * info.num_subcores
    edges_per_w = b // nw
    chunks = edges_per_w // ch
    mesh = plsc.VectorSubcoreMesh(core_axis_name="c", subcore_axis_name="s")

    @functools.partial(
        pl.kernel,
        mesh=mesh,
        out_type=jax.ShapeDtypeStruct((b // 2, dp), jnp.float32),
        scratch_types=[
            pltpu.VMEM((2 * chunks, ch), jnp.int32),
            pltpu.VMEM((ch, dp), jnp.float32),
            pltpu.VMEM((ch, dp), jnp.float32),
            pltpu.VMEM((ch, dp), jnp.float32),
            pltpu.VMEM((ch, dp), jnp.float32),
            pltpu.VMEM((ch // 2, dp), jnp.float32),
            pltpu.VMEM((ch // 2, dp), jnp.float32),
            pltpu.VMEM_SHARED((table.shape[0], dp), jnp.float32),
            pltpu.SemaphoreType.DMA,
            pltpu.SemaphoreType.DMA,
            pltpu.SemaphoreType.DMA,
            pltpu.SemaphoreType.DMA,
            pltpu.SemaphoreType.DMA,
            pltpu.SemaphoreType.DMA,
        ],
    )
    def gather_k(table_hbm, idx_hbm, out_hbm, idx_v, a0, b0, a1, b1,
                 z0, z1, table_sh, ga0, gb0, ga1, gb1, w0, w1):
        wid = lax.axis_index("s") * info.num_cores + lax.axis_index("c")
        zbase = wid * (edges_per_w // 2)
        @pl.when(lax.axis_index("s") == 0)
        def _():
            pltpu.sync_copy(table_hbm, table_sh)
        plsc.subcore_barrier()
        pltpu.sync_copy(idx_hbm.at[pl.ds(wid * 2 * chunks, 2 * chunks)],
                        idx_v)
        sets = ((a0, b0, z0, ga0, gb0, w0), (a1, b1, z1, ga1, gb1, w1))

        def start_gathers(s, abuf, bbuf, sa, sb):
            return (
                pltpu.async_copy(table_sh.at[idx_v.at[s]], abuf, sa),
                pltpu.async_copy(table_sh.at[idx_v.at[chunks + s]],
                                 bbuf, sb),
            )

        def mul_pack(abuf, bbuf, zbuf):
            def row(rr, carry):
                r0 = 2 * rr
                r1 = 2 * rr + 1
                for c in range(_D // 16):
                    sl = pl.ds(c * 16, 16)
                    sh = pl.ds(_D + c * 16, 16)
                    zbuf[rr, sl] = abuf[r0, sl] * bbuf[r0, sl]
                    zbuf[rr, sh] = abuf[r1, sl] * bbuf[r1, sl]
                return carry
            lax.fori_loop(0, ch // 2, row, 0)

        gathers = [None, None]
        writes = [None, None]
        gathers[0] = start_gathers(0, a0, b0, ga0, gb0)
        for s in range(chunks):
            cur = s % 2
            nxt = (s + 1) % 2
            for cp in gathers[cur]:
                cp.wait()
            if s + 1 < chunks:
                gathers[nxt] = start_gathers(s + 1, sets[nxt][0],
                                             sets[nxt][1], sets[nxt][3],
                                             sets[nxt][4])
            if writes[cur] is not None:
                writes[cur].wait()
            mul_pack(sets[cur][0], sets[cur][1], sets[cur][2])
            writes[cur] = pltpu.async_copy(
                sets[cur][2],
                out_hbm.at[pl.ds(zbase + s * (ch // 2), ch // 2)],
                sets[cur][5])
        for w in writes:
            if w is not None:
                w.wait()

    return gather_k(table, idx2d)


def _softmax2_t(v0, v1):
    nrm = jnp.sqrt(v0 * v0 + v1 * v1)
    d = (v1 - v0) / jnp.clip(nrm, 1e-12, None)
    return -jnp.log1p(jnp.exp(d)), -jnp.log1p(jnp.exp(-d))


def _pred_body(z_ref, p1_ref, pb1_ref, p2_ref, pb2_ref, p3_ref,
               pb3_ref, out_ref):
    z = z_ref[...]
    h = jnp.maximum(jnp.dot(z, p1_ref[...],
                            preferred_element_type=jnp.float32)
                    + pb1_ref[...], 0.0)
    h = jnp.maximum(jnp.dot(h, p2_ref[...],
                            preferred_element_type=jnp.float32)
                    + pb2_ref[...], 0.0)
    ot = lax.dot_general(p3_ref[...], h, (((0,), (1,)), ((), ())),
                         preferred_element_type=jnp.float32) + pb3_ref[...]
    e0, e1 = _softmax2_t(ot[0:1, :], ot[1:2, :])
    o0, o1 = _softmax2_t(ot[2:3, :], ot[3:4, :])
    out_ref[...] = jnp.concatenate([e0, e1, o0, o1], axis=0)


def _predictor(z, p1x, pb1x, p2x, pb2x, p3x, pb3x):
    br = 8192
    npair = z.shape[0]
    nblk = npair // br
    dp = z.shape[1]
    return pl.pallas_call(
        _pred_body,
        grid=(nblk,),
        in_specs=[
            pl.BlockSpec((br, dp), lambda i: (i, 0)),
            pl.BlockSpec((dp, dp), lambda i: (0, 0)),
            pl.BlockSpec((1, dp), lambda i: (0, 0)),
            pl.BlockSpec((dp, dp), lambda i: (0, 0)),
            pl.BlockSpec((1, dp), lambda i: (0, 0)),
            pl.BlockSpec((dp, 4), lambda i: (0, 0)),
            pl.BlockSpec((4, 1), lambda i: (0, 0)),
        ],
        out_specs=pl.BlockSpec((4, br), lambda i: (0, i)),
        out_shape=jax.ShapeDtypeStruct((4, npair), jnp.float32),
    )(z, p1x, pb1x, p2x, pb2x, p3x, pb3x)


def kernel(x, adj_t, train_edges, W1, b1, W2, b2, W3, b3,
           P1, pb1, P2, pb2, P3, pb3):
    w3p = jnp.pad(W3, ((0, 0), (0, 128 - W3.shape[1])))
    b3p = jnp.pad(b3.reshape(1, -1), ((0, 0), (0, 128 - b3.shape[0])))
    emb = _gcn_stack(adj_t, x, W1, b1.reshape(1, -1),
                     W2, b2.reshape(1, -1), w3p, b3p)

    info = plsc.get_sparse_core_info()
    nw = info.num_cores * info.num_subcores
    cpw = _NPRED // (nw * 128)
    src3 = train_edges[:, 0].astype(jnp.int32).reshape(nw, cpw, 128)
    dst3 = train_edges[:, 1].astype(jnp.int32).reshape(nw, cpw, 128)
    idx2d = jnp.concatenate([src3, dst3], axis=1).reshape(-1, 128)
    z = _sc_gather_mul(emb, idx2d)

    zd = jnp.zeros((_D, _D), jnp.float32)
    p1x = jnp.block([[P1, zd], [zd, P1]])
    p2x = jnp.block([[P2, zd], [zd, P2]])
    zd2 = jnp.zeros((_D, P3.shape[1]), jnp.float32)
    p3x = jnp.block([[P3, zd2], [zd2, P3]])
    pb1x = jnp.concatenate([pb1, pb1]).reshape(1, -1)
    pb2x = jnp.concatenate([pb2, pb2]).reshape(1, -1)
    pb3x = jnp.concatenate([pb3, pb3]).reshape(-1, 1)
    out4 = _predictor(z, p1x, pb1x, p2x, pb2x, p3x, pb3x)
    out = out4.T.reshape(_NPRED, 2)
    return out[None, :, :]


# speedup vs baseline: 2.0262x; 2.0262x over previous
import functools

import jax
import jax.numpy as jnp
from jax import lax
from jax.experimental import pallas as pl
from jax.experimental.pallas import tpu as pltpu
from jax.experimental.pallas import tpu_sc as plsc

_N = 1024
_D = 64
_NPRED = 131072
_HI = lax.Precision.HIGHEST


def _gcn_body(adj_ref, x_ref, w1_ref, b1_ref, w2_ref, b2_ref,
              w3_ref, b3_ref, out_ref, a_scr):
    adj = adj_ref[...]
    a_scr[...] = ((adj + adj.T) > 0).astype(jnp.float32)
    a = a_scr[...]
    h = x_ref[...]
    for w_ref, b_ref, relu in ((w1_ref, b1_ref, True),
                               (w2_ref, b2_ref, True),
                               (w3_ref, b3_ref, False)):
        hw = jnp.dot(h, w_ref[...],
                     preferred_element_type=jnp.float32)
        o = jnp.dot(a, hw,
                    preferred_element_type=jnp.float32) + b_ref[...]
        h = jnp.maximum(o, 0.0) if relu else o
    out_ref[...] = h


def _gcn_stack(adj, x, w1, b1, w2, b2, w3p, b3p):
    return pl.pallas_call(
        _gcn_body,
        out_shape=jax.ShapeDtypeStruct((_N, 128), jnp.float32),
        scratch_shapes=[pltpu.VMEM((_N, _N), jnp.float32)],
    )(adj, x, w1, b1, w2, b2, w3p, b3p)


def _sc_gather_mul(table, idx2d):
    nrow, ch = idx2d.shape
    b = nrow * ch // 2
    dp = table.shape[1]
    info = plsc.get_sparse_core_info()
    nw = info.num_cores * info.num_subcores
    edges_per_w = b // nw
    chunks = edges_per_w // ch
    mesh = plsc.VectorSubcoreMesh(core_axis_name="c", subcore_axis_name="s")

    @functools.partial(
        pl.kernel,
        mesh=mesh,
        out_type=jax.ShapeDtypeStruct((b, dp), jnp.float32),
        scratch_types=[
            pltpu.VMEM((2 * chunks, ch), jnp.int32),
            pltpu.VMEM((ch, dp), jnp.float32),
            pltpu.VMEM((ch, dp), jnp.float32),
            pltpu.VMEM((ch, dp), jnp.float32),
            pltpu.VMEM((ch, dp), jnp.float32),
            pltpu.VMEM_SHARED((table.shape[0], dp), jnp.float32),
            pltpu.SemaphoreType.DMA,
            pltpu.SemaphoreType.DMA,
            pltpu.SemaphoreType.DMA,
            pltpu.SemaphoreType.DMA,
            pltpu.SemaphoreType.DMA,
            pltpu.SemaphoreType.DMA,
        ],
    )
    def gather_k(table_hbm, idx_hbm, out_hbm, idx_v, a0, b0, a1, b1,
                 table_sh, ga0, gb0, ga1, gb1, w0, w1):
        wid = lax.axis_index("s") * info.num_cores + lax.axis_index("c")
        ebase = wid * edges_per_w
        @pl.when(lax.axis_index("s") == 0)
        def _():
            pltpu.sync_copy(table_hbm, table_sh)
        plsc.subcore_barrier()
        pltpu.sync_copy(idx_hbm.at[pl.ds(wid * 2 * chunks, 2 * chunks)],
                        idx_v)
        sets = ((a0, b0, ga0, gb0, w0), (a1, b1, ga1, gb1, w1))

        def start_gathers(s, abuf, bbuf, sa, sb):
            return (
                pltpu.async_copy(table_sh.at[idx_v.at[s]], abuf, sa),
                pltpu.async_copy(table_sh.at[idx_v.at[chunks + s]],
                                 bbuf, sb),
            )

        def mul_into_a(abuf, bbuf):
            def row(r, carry):
                for c in range(_D // 16):
                    sl = pl.ds(c * 16, 16)
                    abuf[r, sl] = abuf[r, sl] * bbuf[r, sl]
                return carry
            lax.fori_loop(0, ch, row, 0)

        gathers = [None, None]
        writes = [None, None]
        gathers[0] = start_gathers(0, a0, b0, ga0, gb0)
        for s in range(chunks):
            cur = s % 2
            nxt = (s + 1) % 2
            for cp in gathers[cur]:
                cp.wait()
            if s + 1 < chunks:
                if writes[nxt] is not None:
                    writes[nxt].wait()
                gathers[nxt] = start_gathers(s + 1, *sets[nxt][:4])
            mul_into_a(sets[cur][0], sets[cur][1])
            writes[cur] = pltpu.async_copy(
                sets[cur][0],
                out_hbm.at[pl.ds(ebase + s * ch, ch)],
                sets[cur][4])
        for w in writes:
            if w is not None:
                w.wait()

    return gather_k(table, idx2d)


def _pred_body(z_ref, p1_ref, pb1_ref, p2_ref, pb2_ref, p3_ref,
               pb3_ref, out_ref):
    z = z_ref[...]
    h = jnp.maximum(jnp.dot(z, p1_ref[...],
                            preferred_element_type=jnp.float32)
                    + pb1_ref[...], 0.0)
    h = jnp.maximum(jnp.dot(h, p2_ref[...],
                            preferred_element_type=jnp.float32)
                    + pb2_ref[...], 0.0)
    ot = lax.dot_general(p3_ref[...], h, (((0,), (1,)), ((), ())),
                         preferred_element_type=jnp.float32) + pb3_ref[...]
    v0 = ot[0:1, :]
    v1 = ot[1:2, :]
    nrm = jnp.sqrt(v0 * v0 + v1 * v1)
    d = (v1 - v0) / jnp.clip(nrm, 1e-12, None)
    out_ref[...] = jnp.concatenate(
        [-jnp.log1p(jnp.exp(d)), -jnp.log1p(jnp.exp(-d))], axis=0)


def _predictor(z, p1, pb1, p2, pb2, p3, pb3, f_dim):
    br = 8192
    nblk = _NPRED // br
    dp = z.shape[1]
    return pl.pallas_call(
        _pred_body,
        grid=(nblk,),
        in_specs=[
            pl.BlockSpec((br, dp), lambda i: (i, 0)),
            pl.BlockSpec((dp, _D), lambda i: (0, 0)),
            pl.BlockSpec((1, _D), lambda i: (0, 0)),
            pl.BlockSpec((_D, _D), lambda i: (0, 0)),
            pl.BlockSpec((1, _D), lambda i: (0, 0)),
            pl.BlockSpec((_D, f_dim), lambda i: (0, 0)),
            pl.BlockSpec((f_dim, 1), lambda i: (0, 0)),
        ],
        out_specs=pl.BlockSpec((f_dim, br), lambda i: (0, i)),
        out_shape=jax.ShapeDtypeStruct((f_dim, _NPRED), jnp.float32),
    )(z, p1, pb1, p2, pb2, p3, pb3)


def kernel(x, adj_t, train_edges, W1, b1, W2, b2, W3, b3,
           P1, pb1, P2, pb2, P3, pb3):
    w3p = jnp.pad(W3, ((0, 0), (0, 128 - W3.shape[1])))
    b3p = jnp.pad(b3.reshape(1, -1), ((0, 0), (0, 128 - b3.shape[0])))
    emb = _gcn_stack(adj_t, x, W1, b1.reshape(1, -1),
                     W2, b2.reshape(1, -1), w3p, b3p)

    info = plsc.get_sparse_core_info()
    nw = info.num_cores * info.num_subcores
    cpw = _NPRED // (nw * 128)
    src3 = train_edges[:, 0].astype(jnp.int32).reshape(nw, cpw, 128)
    dst3 = train_edges[:, 1].astype(jnp.int32).reshape(nw, cpw, 128)
    idx2d = jnp.concatenate([src3, dst3], axis=1).reshape(-1, 128)
    z = _sc_gather_mul(emb, idx2d)

    f_dim = P3.shape[1]
    p1p = jnp.pad(P1, ((0, z.shape[1] - P1.shape[0]), (0, 0)))
    out_t = _predictor(z, p1p, pb1.reshape(1, -1), P2, pb2.reshape(1, -1),
                       P3, pb3.reshape(-1, 1), f_dim)
    return out_t.T[None, :, :]
